# baseline (speedup 1.0000x reference)
import functools

import jax
import jax.numpy as jnp
from jax import lax
from jax.experimental import pallas as pl
from jax.experimental.pallas import tpu as pltpu
from jax.experimental.pallas import tpu_sc as plsc

_NC = 2
_NS = 16
_NW = _NC * _NS
_LANES = 16


def _sc_embed(comp, lev, t, *, n_comp, n_lev, L, b_per_tile, NB):
    B = comp.shape[0] // L
    D = 1 + n_comp + n_lev
    nchunks = b_per_tile // NB
    rows_per_chunk = NB * L
    mesh = plsc.VectorSubcoreMesh(core_axis_name="c", subcore_axis_name="s")

    @functools.partial(
        pl.kernel,
        out_type=jax.ShapeDtypeStruct((B, L, D), jnp.float32),
        mesh=mesh,
        scratch_types=[
            pltpu.VMEM((NB, L, D), jnp.float32),
            pltpu.VMEM((rows_per_chunk,), jnp.int32),
            pltpu.VMEM((rows_per_chunk,), jnp.int32),
            pltpu.VMEM((rows_per_chunk,), jnp.float32),
        ],
        compiler_params=pltpu.CompilerParams(needs_layout_passes=False),
    )
    def body(comp_hbm, lev_hbm, t_hbm, out_hbm, stage, comp_v, lev_v, t_v):
        wid = lax.axis_index("s") * _NC + lax.axis_index("c")
        tile_b0 = wid * b_per_tile
        ones = jnp.full((_LANES,), 1.0, jnp.float32)
        zeros = jnp.zeros((_LANES,), jnp.float32)
        lane = lax.iota(jnp.int32, _LANES)
        def zero_row(r, carry):
            b = r // L
            l = r - b * L

            def zero_seg(s, carry2):
                stage[b, l, pl.ds(s * _LANES, _LANES)] = zeros
                return carry2

            lax.fori_loop(0, D // _LANES, zero_seg, 0)
            stage[b, l, pl.ds(D - _LANES, _LANES)] = zeros
            return carry

        lax.fori_loop(0, rows_per_chunk, zero_row, 0)

        def chunk_body(c, carry):
            b0 = tile_b0 + c * NB
            r_base = b0 * L
            pltpu.sync_copy(comp_hbm.at[pl.ds(r_base, rows_per_chunk)], comp_v)
            pltpu.sync_copy(lev_hbm.at[pl.ds(r_base, rows_per_chunk)], lev_v)
            pltpu.sync_copy(t_hbm.at[pl.ds(r_base, rows_per_chunk)], t_v)

            def group(g, carry2):
                r0 = g * _LANES
                r16 = lane + r0
                b16 = r16 // L
                l16 = r16 - b16 * L
                c16 = comp_v[pl.ds(r0, _LANES)]
                v16 = lev_v[pl.ds(r0, _LANES)]
                t16 = t_v[pl.ds(r0, _LANES)]
                zch = jnp.zeros((_LANES,), jnp.int32)
                plsc.store_scatter(stage, [b16, l16, zch], t16)
                plsc.store_scatter(stage, [b16, l16, 1 + c16], ones,
                                   mask=c16 < n_comp)
                plsc.store_scatter(stage, [b16, l16, (1 + n_comp) + v16], ones,
                                   mask=v16 < n_lev)
                return carry2

            lax.fori_loop(0, rows_per_chunk // _LANES, group, 0)
            pltpu.sync_copy(stage, out_hbm.at[pl.ds(b0, NB)])

            def ungroup(g, carry2):
                r0 = g * _LANES
                r16 = lane + r0
                b16 = r16 // L
                l16 = r16 - b16 * L
                c16 = comp_v[pl.ds(r0, _LANES)]
                v16 = lev_v[pl.ds(r0, _LANES)]
                plsc.store_scatter(stage, [b16, l16, 1 + c16], zeros,
                                   mask=c16 < n_comp)
                plsc.store_scatter(stage, [b16, l16, (1 + n_comp) + v16], zeros,
                                   mask=v16 < n_lev)
                return carry2

            lax.fori_loop(0, rows_per_chunk // _LANES, ungroup, 0)
            return carry

        lax.fori_loop(0, nchunks, chunk_body, 0)

    return body(comp, lev, t)


def kernel(components, levels, time_elapsed, comp_table, level_table):
    n_comp = comp_table.shape[1]
    n_lev = level_table.shape[1]
    B, L = components.shape
    N = B * L

    comp = components.reshape(N).astype(jnp.int32)
    lev = levels.reshape(N).astype(jnp.int32)
    t = time_elapsed.reshape(N)

    b_per_tile = B // _NW
    NB = 8
    assert b_per_tile % NB == 0
    assert (NB * L) % _LANES == 0

    return _sc_embed(comp, lev, t, n_comp=n_comp, n_lev=n_lev, L=L,
                     b_per_tile=b_per_tile, NB=NB)


# speedup vs baseline: 3.2131x; 3.2131x over previous
import functools

import jax
import jax.numpy as jnp
from jax import lax
from jax.experimental import pallas as pl
from jax.experimental.pallas import tpu as pltpu
from jax.experimental.pallas import tpu_sc as plsc

_NC = 2
_NS = 16
_NW = _NC * _NS
_LANES = 16
_BC = 128
_DPAD = 200


def _sc_embed(comp_t, lev_t, t_t, *, n_comp, n_lev, L, B, NL):
    D = 1 + n_comp + n_lev
    per_tile_in = L * _BC
    nchunks = L // NL
    groups = _BC // _LANES
    mesh = plsc.VectorSubcoreMesh(core_axis_name="c", subcore_axis_name="s",
                                  num_cores=_NC, num_subcores=_NS)

    @functools.partial(
        pl.kernel,
        out_type=jax.ShapeDtypeStruct((L, D, B), jnp.float32),
        mesh=mesh,
        scratch_types=[
            pltpu.VMEM((NL, _DPAD, _BC), jnp.float32),
            pltpu.VMEM((per_tile_in,), jnp.int32),
            pltpu.VMEM((per_tile_in,), jnp.int32),
            pltpu.VMEM((per_tile_in,), jnp.float32),
        ],
        compiler_params=pltpu.CompilerParams(needs_layout_passes=False),
    )
    def body(comp_hbm, lev_hbm, t_hbm, out_hbm, stage, comp_v, lev_v, t_v):
        wid = lax.axis_index("s") * _NC + lax.axis_index("c")
        b0 = pl.multiple_of(wid * _BC, _BC)
        ones = jnp.full((_LANES,), 1.0, jnp.float32)
        zeros = jnp.zeros((_LANES,), jnp.float32)
        lane = lax.iota(jnp.int32, _LANES)

        in0 = wid * per_tile_in
        pltpu.sync_copy(comp_hbm.at[pl.ds(in0, per_tile_in)], comp_v)
        pltpu.sync_copy(lev_hbm.at[pl.ds(in0, per_tile_in)], lev_v)
        pltpu.sync_copy(t_hbm.at[pl.ds(in0, per_tile_in)], t_v)

        def zero_flat(i, carry):
            r = i * _LANES + lane
            li = r // (_DPAD * _BC)
            rem = r - li * (_DPAD * _BC)
            ch = rem // _BC
            bi = rem - ch * _BC
            plsc.store_scatter(stage, [li, ch, bi], zeros)
            return carry
        lax.fori_loop(0, (NL * _DPAD * _BC) // _LANES, zero_flat, 0)

        def chunk_body(c, carry):
            l0 = c * NL
            for li in range(NL):
                lconst = jnp.full((_LANES,), li, jnp.int32)
                row_in = (l0 + li) * _BC

                def group(g, carry2):
                    o = row_in + g * _LANES
                    c16 = comp_v[pl.ds(o, _LANES)]
                    v16 = lev_v[pl.ds(o, _LANES)]
                    t16 = t_v[pl.ds(o, _LANES)]
                    b16 = lane + g * _LANES
                    zch = jnp.zeros((_LANES,), jnp.int32)
                    plsc.store_scatter(stage, [lconst, zch, b16], t16)
                    plsc.store_scatter(stage, [lconst, 1 + c16, b16], ones,
                                       mask=c16 < n_comp)
                    plsc.store_scatter(stage, [lconst, (1 + n_comp) + v16, b16],
                                       ones, mask=v16 < n_lev)
                    return carry2

                lax.fori_loop(0, groups, group, 0)

            pltpu.sync_copy(stage.at[:, pl.ds(0, D), :],
                            out_hbm.at[pl.ds(l0, NL), :, pl.ds(b0, _BC)])

            for li in range(NL):
                lconst = jnp.full((_LANES,), li, jnp.int32)
                row_in = (l0 + li) * _BC

                def ungroup(g, carry2):
                    o = row_in + g * _LANES
                    c16 = comp_v[pl.ds(o, _LANES)]
                    v16 = lev_v[pl.ds(o, _LANES)]
                    b16 = lane + g * _LANES
                    plsc.store_scatter(stage, [lconst, 1 + c16, b16], zeros,
                                       mask=c16 < n_comp)
                    plsc.store_scatter(stage, [lconst, (1 + n_comp) + v16, b16],
                                       zeros, mask=v16 < n_lev)
                    return carry2

                lax.fori_loop(0, groups, ungroup, 0)
            return carry

        lax.fori_loop(0, nchunks, chunk_body, 0)

    return body(comp_t, lev_t, t_t)


def kernel(components, levels, time_elapsed, comp_table, level_table):
    n_comp = comp_table.shape[1]
    n_lev = level_table.shape[1]
    D = 1 + n_comp + n_lev
    B, L = components.shape

    def _prep(x, dtype):
        xt = x.astype(dtype).T.reshape(L, _NW, _BC)
        return xt.transpose(1, 0, 2).reshape(-1)

    comp_t = _prep(components, jnp.int32)
    lev_t = _prep(levels, jnp.int32)
    t_t = _prep(time_elapsed, jnp.float32)

    NL = 2
    assert L % NL == 0 and B % (_NW * _BC) == 0

    out_t = _sc_embed(comp_t, lev_t, t_t, n_comp=n_comp, n_lev=n_lev,
                      L=L, B=B, NL=NL)
    return out_t.transpose(2, 0, 1)


# speedup vs baseline: 4.1132x; 1.2801x over previous
import functools

import jax
import jax.numpy as jnp
from jax import lax
from jax.experimental import pallas as pl
from jax.experimental.pallas import tpu as pltpu
from jax.experimental.pallas import tpu_sc as plsc

_NC = 2
_NS = 16
_NW = _NC * _NS
_LANES = 16
_BC = 128
_DPAD = 200


def _sc_embed(comp_t, lev_t, t_t, *, n_comp, n_lev, L, B):
    D = 1 + n_comp + n_lev
    per_tile_in = L * _BC
    groups = _BC // _LANES
    mesh = plsc.VectorSubcoreMesh(core_axis_name="c", subcore_axis_name="s",
                                  num_cores=_NC, num_subcores=_NS)

    @functools.partial(
        pl.kernel,
        out_type=jax.ShapeDtypeStruct((L, D, B), jnp.float32),
        mesh=mesh,
        scratch_types=[
            pltpu.VMEM((1, _DPAD, _BC), jnp.float32),
            pltpu.VMEM((1, _DPAD, _BC), jnp.float32),
            pltpu.VMEM((per_tile_in,), jnp.int32),
            pltpu.VMEM((per_tile_in,), jnp.int32),
            pltpu.VMEM((per_tile_in,), jnp.float32),
            pltpu.SemaphoreType.DMA,
            pltpu.SemaphoreType.DMA,
        ],
        compiler_params=pltpu.CompilerParams(needs_layout_passes=False),
    )
    def body(comp_hbm, lev_hbm, t_hbm, out_hbm,
             stage0, stage1, comp_v, lev_v, t_v, sem0, sem1):
        wid = lax.axis_index("s") * _NC + lax.axis_index("c")
        b0 = pl.multiple_of(wid * _BC, _BC)
        ones = jnp.full((_LANES,), 1.0, jnp.float32)
        zeros = jnp.zeros((_LANES,), jnp.float32)
        zrow = jnp.zeros((_LANES,), jnp.int32)
        lane = lax.iota(jnp.int32, _LANES)
        stages = (stage0, stage1)
        sems = (sem0, sem1)

        in0 = wid * per_tile_in
        pltpu.sync_copy(comp_hbm.at[pl.ds(in0, per_tile_in)], comp_v)
        pltpu.sync_copy(lev_hbm.at[pl.ds(in0, per_tile_in)], lev_v)
        pltpu.sync_copy(t_hbm.at[pl.ds(in0, per_tile_in)], t_v)

        def zero_flat(i, carry):
            r = i * _LANES + lane
            ch = r // _BC
            bi = r - ch * _BC
            plsc.store_scatter(stage0, [zrow, ch, bi], zeros)
            plsc.store_scatter(stage1, [zrow, ch, bi], zeros)
            return carry

        lax.fori_loop(0, (_DPAD * _BC) // _LANES, zero_flat, 0)

        def scatter(stage, l, value_t):
            row_in = l * _BC

            def group(g, carry):
                o = row_in + g * _LANES
                c16 = comp_v[pl.ds(o, _LANES)]
                v16 = lev_v[pl.ds(o, _LANES)]
                b16 = lane + g * _LANES
                if value_t:
                    t16 = t_v[pl.ds(o, _LANES)]
                    plsc.store_scatter(stage, [zrow, zrow, b16], t16)
                val = ones if value_t else zeros
                plsc.store_scatter(stage, [zrow, 1 + c16, b16], val,
                                   mask=c16 < n_comp)
                plsc.store_scatter(stage, [zrow, (1 + n_comp) + v16, b16],
                                   val, mask=v16 < n_lev)
                return carry

            lax.fori_loop(0, groups, group, 0)

        def start(stage, sem, l):
            return pltpu.async_copy(
                stage.at[:, pl.ds(0, D), :],
                out_hbm.at[pl.ds(l, 1), :, pl.ds(b0, _BC)], sem)

        def wait(stage, sem, l):
            pltpu.make_async_copy(
                stage.at[:, pl.ds(0, D), :],
                out_hbm.at[pl.ds(l, 1), :, pl.ds(b0, _BC)], sem).wait()

        for li in range(2):
            scatter(stages[li], li, True)
            start(stages[li], sems[li], li)

        def pair(cc, carry):
            for li in range(2):
                l = cc * 2 + li
                wait(stages[li], sems[li], l - 2)
                scatter(stages[li], l - 2, False)
                scatter(stages[li], l, True)
                start(stages[li], sems[li], l)
            return carry

        lax.fori_loop(1, L // 2, pair, 0)

        for li in range(2):
            wait(stages[li], sems[li], L - 2 + li)

    return body(comp_t, lev_t, t_t)


def kernel(components, levels, time_elapsed, comp_table, level_table):
    n_comp = comp_table.shape[1]
    n_lev = level_table.shape[1]
    B, L = components.shape

    def _prep(x, dtype):
        xt = x.astype(dtype).T.reshape(L, _NW, _BC)
        return xt.transpose(1, 0, 2).reshape(-1)

    comp_t = _prep(components, jnp.int32)
    lev_t = _prep(levels, jnp.int32)
    t_t = _prep(time_elapsed, jnp.float32)

    assert L % 2 == 0 and B % (_NW * _BC) == 0

    out_t = _sc_embed(comp_t, lev_t, t_t, n_comp=n_comp, n_lev=n_lev,
                      L=L, B=B)
    return out_t.transpose(2, 0, 1)
